# Initial kernel scaffold; baseline (speedup 1.0000x reference)
import functools

import jax
import jax.numpy as jnp
from jax import lax
from jax.experimental import pallas as pl
from jax.experimental.pallas import tpu as pltpu
from jax.experimental.pallas import tpu_sc as plsc

_BATCH = 16384
_NF = 26
_D = 32
_B = _BATCH * _NF

_info = plsc.get_sparse_core_info()
_NC, _NS = _info.num_cores, _info.num_subcores
_NW = _NC * _NS
_BPW = _B // _NW
_CH = 1024
_NCHUNK = _BPW // _CH

_mesh = plsc.VectorSubcoreMesh(core_axis_name="c", subcore_axis_name="s")


@functools.partial(
    pl.kernel,
    mesh=_mesh,
    out_type=jax.ShapeDtypeStruct((_B, _D), jnp.float32),
    scratch_types=[
        pltpu.VMEM((_NCHUNK, _CH), jnp.int32),
        pltpu.VMEM((2, _CH, _D), jnp.float32),
        pltpu.SemaphoreType.DMA,
    ],
)
def _emb_lookup(x_hbm, w_hbm, out_hbm, idx_v, rows_v, gsem):
    wid = lax.axis_index("s") * _NC + lax.axis_index("c")
    base = wid * _BPW
    pltpu.sync_copy(x_hbm.at[wid], idx_v)
    gathers = [None] * _NCHUNK
    gathers[0] = pltpu.async_copy(w_hbm.at[idx_v.at[0]], rows_v.at[0], gsem)
    for i in range(_NCHUNK):
        if i + 1 < _NCHUNK:
            gathers[i + 1] = pltpu.async_copy(
                w_hbm.at[idx_v.at[i + 1]], rows_v.at[(i + 1) % 2], gsem)
        gathers[i].wait()
        pltpu.sync_copy(rows_v.at[i % 2],
                        out_hbm.at[pl.ds(base + i * _CH, _CH)])


def kernel(x, weight):
    xf = x.astype(jnp.int32).reshape(_NW, _NCHUNK, _CH)
    out = _emb_lookup(xf, weight)
    return out.reshape(_BATCH, _NF, _D)


# speedup vs baseline: 1.5765x; 1.5765x over previous
import functools

import jax
import jax.numpy as jnp
from jax import lax
from jax.experimental import pallas as pl
from jax.experimental.pallas import tpu as pltpu
from jax.experimental.pallas import tpu_sc as plsc

_BATCH = 16384
_NF = 26
_D = 32
_B = _BATCH * _NF

_info = plsc.get_sparse_core_info()
_NC, _NS = _info.num_cores, _info.num_subcores
_NW = _NC * _NS
_BPW = _B // _NW
_CH = 1024
_NCHUNK = _BPW // _CH

_mesh = plsc.VectorSubcoreMesh(core_axis_name="c", subcore_axis_name="s")


@functools.partial(
    pl.kernel,
    mesh=_mesh,
    out_type=jax.ShapeDtypeStruct((_B, _D), jnp.float32),
    scratch_types=[
        pltpu.VMEM((_BPW,), jnp.int32),
        pltpu.VMEM((2, _CH, _D), jnp.float32),
        pltpu.SemaphoreType.DMA,
    ],
    compiler_params=pltpu.CompilerParams(use_tc_tiling_on_sc=False),
)
def _emb_lookup(x_hbm, w_hbm, out_hbm, idx_v, rows_v, gsem):
    wid = lax.axis_index("s") * _NC + lax.axis_index("c")
    base = wid * _BPW
    pltpu.sync_copy(x_hbm.at[pl.ds(base, _BPW)], idx_v)
    gathers = [None] * _NCHUNK
    gathers[0] = pltpu.async_copy(
        w_hbm.at[idx_v.at[pl.ds(0, _CH)]], rows_v.at[0], gsem)
    for i in range(_NCHUNK):
        if i + 1 < _NCHUNK:
            gathers[i + 1] = pltpu.async_copy(
                w_hbm.at[idx_v.at[pl.ds((i + 1) * _CH, _CH)]],
                rows_v.at[(i + 1) % 2], gsem)
        gathers[i].wait()
        pltpu.sync_copy(rows_v.at[i % 2],
                        out_hbm.at[pl.ds(base + i * _CH, _CH)])


def kernel(x, weight):
    xf = x.astype(jnp.int32).reshape(_B)
    out = _emb_lookup(xf, weight)
    return out.reshape(_BATCH, _NF, _D)


# speedup vs baseline: 1.5773x; 1.0005x over previous
import functools

import jax
import jax.numpy as jnp
from jax import lax
from jax.experimental import pallas as pl
from jax.experimental.pallas import tpu as pltpu
from jax.experimental.pallas import tpu_sc as plsc

_BATCH = 16384
_NF = 26
_D = 32
_B = _BATCH * _NF

_info = plsc.get_sparse_core_info()
_NC, _NS = _info.num_cores, _info.num_subcores
_NW = _NC * _NS
_BPW = _B // _NW
_CH = 832
_NCHUNK = _BPW // _CH
_NBUF = 4

_mesh = plsc.VectorSubcoreMesh(core_axis_name="c", subcore_axis_name="s")


@functools.partial(
    pl.kernel,
    mesh=_mesh,
    out_type=jax.ShapeDtypeStruct((_B, _D), jnp.float32),
    scratch_types=[
        pltpu.VMEM((_BPW,), jnp.int32),
        pltpu.VMEM((_NBUF, _CH, _D), jnp.float32),
        pltpu.SemaphoreType.DMA,
        pltpu.SemaphoreType.DMA,
    ],
    compiler_params=pltpu.CompilerParams(use_tc_tiling_on_sc=False),
)
def _emb_lookup(x_hbm, w_hbm, out_hbm, idx_v, rows_v, gsem, osem):
    wid = lax.axis_index("s") * _NC + lax.axis_index("c")
    base = wid * _BPW
    pltpu.sync_copy(x_hbm.at[pl.ds(base, _BPW)], idx_v)

    def gather(j):
        return pltpu.async_copy(
            w_hbm.at[idx_v.at[pl.ds(j * _CH, _CH)]],
            rows_v.at[j % _NBUF], gsem)

    gathers = [None] * _NCHUNK
    outs = [None] * _NCHUNK
    for j in range(min(_NBUF - 1, _NCHUNK)):
        gathers[j] = gather(j)
    for i in range(_NCHUNK):
        gathers[i].wait()
        outs[i] = pltpu.async_copy(
            rows_v.at[i % _NBUF], out_hbm.at[pl.ds(base + i * _CH, _CH)],
            osem)
        j = i + _NBUF - 1
        if j < _NCHUNK:
            if j - _NBUF >= 0:
                outs[j - _NBUF].wait()
            gathers[j] = gather(j)
    for i in range(max(0, _NCHUNK - _NBUF), _NCHUNK):
        outs[i].wait()


def kernel(x, weight):
    xf = x.astype(jnp.int32).reshape(_B)
    out = _emb_lookup(xf, weight)
    return out.reshape(_BATCH, _NF, _D)
